# Initial kernel scaffold; baseline (speedup 1.0000x reference)
import jax
import jax.numpy as jnp
from jax.experimental import pallas as pl


def kernel(g, features, W1, W2, att_src, att_dst):
    raise NotImplementedError("write your pallas kernel here")


# speedup vs baseline: 16.2662x; 16.2662x over previous
import functools

import jax
import jax.numpy as jnp
from jax import lax
from jax.experimental import pallas as pl
from jax.experimental.pallas import tpu as pltpu
from jax.experimental.pallas import tpu_sc as plsc

N = 10000
E = 320000
IN_DIM, HID, OUT = 128, 64, 32
NC, NS = 2, 16
NW = NC * NS
EPW = E // NW
C = 80
NCHUNK = EPW // C
RPT = N // NS
BLK = 1000

_mesh = plsc.VectorSubcoreMesh(core_axis_name="c", subcore_axis_name="s")
_sc_params = pltpu.CompilerParams(use_tc_tiling_on_sc=False)


def _tc1_body(x_ref, w1_ref, as_ref, ad_ref, h_ref, asrc_ref, adst_ref):
    x = x_ref[...]
    h = lax.dot_general(x, w1_ref[...], (((1,), (1,)), ((), ())),
                        preferred_element_type=jnp.float32)
    h_ref[...] = h
    asrc_ref[...] = jnp.sum(h * as_ref[...][None, :], axis=1)
    adst_ref[...] = jnp.sum(h * ad_ref[...][None, :], axis=1)


def _tc1(x, w1, att_src, att_dst):
    return pl.pallas_call(
        _tc1_body,
        out_shape=[
            jax.ShapeDtypeStruct((N, HID), jnp.float32),
            jax.ShapeDtypeStruct((N,), jnp.float32),
            jax.ShapeDtypeStruct((N,), jnp.float32),
        ],
    )(x, w1, att_src, att_dst)


@functools.partial(
    pl.kernel,
    out_type=[
        jax.ShapeDtypeStruct((E,), jnp.float32),
        jax.ShapeDtypeStruct((NC, N, HID), jnp.float32),
        jax.ShapeDtypeStruct((NC * N,), jnp.float32),
    ],
    mesh=_mesh,
    scratch_types=[
        pltpu.VMEM((C,), jnp.int32),
        pltpu.VMEM((C,), jnp.int32),
        pltpu.VMEM((C, HID), jnp.float32),
        pltpu.VMEM((C,), jnp.float32),
        pltpu.VMEM((C,), jnp.float32),
        pltpu.VMEM((C,), jnp.float32),
        pltpu.VMEM((BLK, HID), jnp.float32),
        pltpu.VMEM((BLK,), jnp.float32),
        pltpu.VMEM_SHARED((N, HID), jnp.float32),
        pltpu.VMEM_SHARED((N,), jnp.float32),
        pltpu.VMEM_SHARED((N,), jnp.float32),
        pltpu.VMEM_SHARED((N,), jnp.float32),
    ],
    compiler_params=_sc_params,
)
def _sc_a(src_hbm, dst_hbm, asrc_hbm, adst_hbm, h_hbm, z64_hbm, z1_hbm,
          ex_hbm, u_hbm, s_hbm,
          sidx, didx, rows, asb, adb, exb, stg, stg1,
          u_sh, s_sh, asrc_sh, adst_sh):
    c = lax.axis_index("c")
    tid = lax.axis_index("s")
    wid = tid * NC + c
    r0 = tid * BLK
    @pl.when(tid < 10)
    def _():
        pltpu.sync_copy(z64_hbm.at[pl.ds(r0, BLK), :], stg)
        pltpu.sync_copy(stg, u_sh.at[pl.ds(r0, BLK), :])
        pltpu.sync_copy(z1_hbm.at[pl.ds(r0, BLK)], stg1)
        pltpu.sync_copy(stg1, s_sh.at[pl.ds(r0, BLK)])
        pltpu.sync_copy(asrc_hbm.at[pl.ds(r0, BLK)], stg1)
        pltpu.sync_copy(stg1, asrc_sh.at[pl.ds(r0, BLK)])
        pltpu.sync_copy(adst_hbm.at[pl.ds(r0, BLK)], stg1)
        pltpu.sync_copy(stg1, adst_sh.at[pl.ds(r0, BLK)])

    plsc.subcore_barrier()

    ebase = wid * EPW

    def chunk(i, carry):
        base = ebase + i * C
        pltpu.sync_copy(src_hbm.at[pl.ds(base, C)], sidx)
        pltpu.sync_copy(dst_hbm.at[pl.ds(base, C)], didx)
        pltpu.sync_copy(h_hbm.at[sidx], rows)
        pltpu.sync_copy(asrc_sh.at[sidx], asb)
        pltpu.sync_copy(adst_sh.at[didx], adb)
        for j in range(C // 16):
            av = asb[pl.ds(j * 16, 16)] + adb[pl.ds(j * 16, 16)]
            e = jnp.where(av >= 0.0, av, jnp.float32(0.2) * av)
            ex16 = jnp.exp(e)
            exb[pl.ds(j * 16, 16)] = ex16
            for k in range(16):
                xs = ex16[k]
                r = j * 16 + k
                for q in range(HID // 16):
                    rows[r, pl.ds(q * 16, 16)] = (
                        rows[r, pl.ds(q * 16, 16)] * xs)
        pltpu.sync_copy(exb, ex_hbm.at[pl.ds(base, C)])
        pltpu.sync_copy(rows, u_sh.at[didx], add=True)
        pltpu.sync_copy(exb, s_sh.at[didx], add=True)
        return carry

    lax.fori_loop(0, NCHUNK, chunk, 0)
    plsc.subcore_barrier()

    @pl.when(tid < 10)
    def _():
        pltpu.sync_copy(u_sh.at[pl.ds(r0, BLK), :], stg)
        pltpu.sync_copy(stg, u_hbm.at[c, pl.ds(r0, BLK), :])
        pltpu.sync_copy(s_sh.at[pl.ds(r0, BLK)], stg1)
        pltpu.sync_copy(stg1, s_hbm.at[pl.ds(c * N + r0, BLK)])


def _elu(x):
    return jnp.where(x > 0.0, x, jnp.exp(jnp.minimum(x, 0.0)) - 1.0)


def _tc2_body(u_ref, s_ref, w2_ref, h2_ref, m3_ref):
    u = u_ref[...]
    s = s_ref[...]
    den = s[0] + s[1] + jnp.float32(1e-16)
    h1 = _elu((u[0] + u[1]) / den)
    h2 = lax.dot_general(h1, w2_ref[...], (((1,), (1,)), ((), ())),
                         preferred_element_type=jnp.float32)
    h2_ref[...] = h2
    m3_ref[...] = lax.dot_general(h2, w2_ref[...], (((1,), (0,)), ((), ())),
                                  preferred_element_type=jnp.float32)


def _tc2(u, s, w2):
    return pl.pallas_call(
        _tc2_body,
        out_shape=[
            jax.ShapeDtypeStruct((N, OUT), jnp.float32),
            jax.ShapeDtypeStruct((N, HID), jnp.float32),
        ],
    )(u, s, w2)


@functools.partial(
    pl.kernel,
    out_type=[
        jax.ShapeDtypeStruct((NC, N, HID), jnp.float32),
    ],
    mesh=_mesh,
    scratch_types=[
        pltpu.VMEM((C,), jnp.int32),
        pltpu.VMEM((C,), jnp.int32),
        pltpu.VMEM((C, HID), jnp.float32),
        pltpu.VMEM((C,), jnp.float32),
        pltpu.VMEM((BLK, HID), jnp.float32),
        pltpu.VMEM_SHARED((N, HID), jnp.float32),
    ],
    compiler_params=_sc_params,
)
def _sc_b(src_hbm, dst_hbm, ex_hbm, m3_hbm, z64_hbm,
          v_hbm,
          sidx, didx, rows, exb, stg, v_sh):
    c = lax.axis_index("c")
    tid = lax.axis_index("s")
    wid = tid * NC + c
    r0 = tid * BLK

    @pl.when(tid < 10)
    def _():
        pltpu.sync_copy(z64_hbm.at[pl.ds(r0, BLK), :], stg)
        pltpu.sync_copy(stg, v_sh.at[pl.ds(r0, BLK), :])

    plsc.subcore_barrier()

    ebase = wid * EPW

    def chunk(i, carry):
        base = ebase + i * C
        pltpu.sync_copy(src_hbm.at[pl.ds(base, C)], sidx)
        pltpu.sync_copy(dst_hbm.at[pl.ds(base, C)], didx)
        pltpu.sync_copy(ex_hbm.at[pl.ds(base, C)], exb)
        pltpu.sync_copy(m3_hbm.at[sidx], rows)
        for j in range(C // 16):
            ex16 = exb[pl.ds(j * 16, 16)]
            for k in range(16):
                xs = ex16[k]
                r = j * 16 + k
                for q in range(HID // 16):
                    rows[r, pl.ds(q * 16, 16)] = (
                        rows[r, pl.ds(q * 16, 16)] * xs)
        pltpu.sync_copy(rows, v_sh.at[didx], add=True)
        return carry

    lax.fori_loop(0, NCHUNK, chunk, 0)
    plsc.subcore_barrier()

    @pl.when(tid < 10)
    def _():
        pltpu.sync_copy(v_sh.at[pl.ds(r0, BLK), :], stg)
        pltpu.sync_copy(stg, v_hbm.at[c, pl.ds(r0, BLK), :])


def _tc3_body(v_ref, s_ref, w1_ref, h4_ref):
    v = v_ref[...]
    s = s_ref[...]
    den = s[0] + s[1] + jnp.float32(1e-16)
    h3 = _elu((v[0] + v[1]) / den)
    h4_ref[...] = lax.dot_general(h3, w1_ref[...], (((1,), (0,)), ((), ())),
                                  preferred_element_type=jnp.float32)


def _tc3(v, s, w1):
    return pl.pallas_call(
        _tc3_body,
        out_shape=jax.ShapeDtypeStruct((N, IN_DIM), jnp.float32),
    )(v, s, w1)


def kernel(g, features, W1, W2, att_src, att_dst):
    src = g[0].astype(jnp.int32)
    dst = g[1].astype(jnp.int32)
    z64 = jnp.zeros((N, HID), jnp.float32)
    z1 = jnp.zeros((N,), jnp.float32)

    h, asrc, adst = _tc1(features, W1, att_src, att_dst)
    ex, u, s = _sc_a(src, dst, asrc, adst, h, z64, z1)
    s3 = s.reshape(NC, N, 1)
    h2, m3 = _tc2(u, s3, W2)
    (v,) = _sc_b(src, dst, ex, m3, z64)
    h4 = _tc3(v, s3, W1)
    return (h2, h4)


# speedup vs baseline: 20.6019x; 1.2665x over previous
import functools

import jax
import jax.numpy as jnp
from jax import lax
from jax.experimental import pallas as pl
from jax.experimental.pallas import tpu as pltpu
from jax.experimental.pallas import tpu_sc as plsc

N = 10000
E = 320000
IN_DIM, HID, OUT = 128, 64, 32
HHALF = HID // 2
NC, NS = 2, 16
EPT = E // NS
C = 128
EPTP = ((EPT + C - 1) // C) * C
PAD = EPTP - EPT
NCH = EPTP // C
NP = N + 16
BLK = 1000

_mesh = plsc.VectorSubcoreMesh(core_axis_name="c", subcore_axis_name="s")
_sc_params = pltpu.CompilerParams(use_tc_tiling_on_sc=False)


def _tc1_body(x_ref, w1_ref, as_ref, ad_ref, hsp_ref, asrc_ref, adst_ref):
    x = x_ref[...]
    h = lax.dot_general(x, w1_ref[...], (((1,), (1,)), ((), ())),
                        preferred_element_type=jnp.float32)
    hsp_ref[...] = jnp.concatenate([h[:, :HHALF], h[:, HHALF:]], axis=0)
    asrc_ref[...] = jnp.sum(h * as_ref[...][None, :], axis=1)
    adst_ref[...] = jnp.sum(h * ad_ref[...][None, :], axis=1)


def _tc1(x, w1, att_src, att_dst):
    return pl.pallas_call(
        _tc1_body,
        out_shape=[
            jax.ShapeDtypeStruct((NC * N, HHALF), jnp.float32),
            jax.ShapeDtypeStruct((N,), jnp.float32),
            jax.ShapeDtypeStruct((N,), jnp.float32),
        ],
    )(x, w1, att_src, att_dst)


@functools.partial(
    pl.kernel,
    out_type=[
        jax.ShapeDtypeStruct((NS * EPTP,), jnp.float32),
        jax.ShapeDtypeStruct((NC * N, HHALF), jnp.float32),
        jax.ShapeDtypeStruct((NC * N,), jnp.float32),
    ],
    mesh=_mesh,
    scratch_types=[
        pltpu.VMEM((EPTP,), jnp.int32),
        pltpu.VMEM((EPTP,), jnp.int32),
        pltpu.VMEM((EPTP,), jnp.float32),
        pltpu.VMEM((C,), jnp.int32),
        pltpu.VMEM((C,), jnp.int32),
        pltpu.VMEM((C, HHALF), jnp.float32),
        pltpu.VMEM((C,), jnp.float32),
        pltpu.VMEM((C,), jnp.float32),
        pltpu.VMEM((BLK, HHALF), jnp.float32),
        pltpu.VMEM((BLK,), jnp.float32),
        pltpu.VMEM_SHARED((NP, HHALF), jnp.float32),
        pltpu.VMEM_SHARED((NP,), jnp.float32),
        pltpu.VMEM_SHARED((NC * N,), jnp.float32),
        pltpu.VMEM_SHARED((NP,), jnp.float32),
    ],
    compiler_params=_sc_params,
)
def _sc_a(src_hbm, dst_hbm, asrc_hbm, adst_hbm, hsp_hbm, z32_hbm, z1_hbm,
          ex_hbm, u_hbm, s_hbm,
          sidx_all, didx_all, exv_all, sidx, didx, rows, asb, adb,
          stg, stg1, u_sh, s_sh, asrc_sh, adst_sh):
    c = lax.axis_index("c")
    tid = lax.axis_index("s")
    r0 = tid * BLK
    @pl.when(tid < 10)
    def _():
        pltpu.sync_copy(z32_hbm.at[pl.ds(r0, BLK), :], stg)
        pltpu.sync_copy(stg, u_sh.at[pl.ds(r0, BLK), :])
        pltpu.sync_copy(z1_hbm.at[pl.ds(r0, BLK)], stg1)
        pltpu.sync_copy(stg1, s_sh.at[pl.ds(r0, BLK)])
        pltpu.sync_copy(asrc_hbm.at[pl.ds(r0, BLK)], stg1)
        pltpu.sync_copy(stg1, asrc_sh.at[pl.ds(r0, BLK)])
        pltpu.sync_copy(stg1, asrc_sh.at[pl.ds(N + r0, BLK)])
        pltpu.sync_copy(adst_hbm.at[pl.ds(r0, BLK)], stg1)
        pltpu.sync_copy(stg1, adst_sh.at[pl.ds(r0, BLK)])

    @pl.when(tid == 10)
    def _():
        pltpu.sync_copy(z1_hbm.at[pl.ds(0, 16)], stg1.at[pl.ds(0, 16)])
        pltpu.sync_copy(stg1.at[pl.ds(0, 16)], adst_sh.at[pl.ds(N, 16)])

    plsc.subcore_barrier()

    ebase = tid * EPTP
    pltpu.sync_copy(src_hbm.at[pl.ds(ebase, EPTP)], sidx_all)
    pltpu.sync_copy(dst_hbm.at[pl.ds(ebase, EPTP)], didx_all)
    coff = c * N

    def chunk(i, carry):
        off = i * C
        for q in range(C // 16):
            s16 = sidx_all[pl.ds(off + q * 16, 16)]
            sidx[pl.ds(q * 16, 16)] = s16 + coff
            didx[pl.ds(q * 16, 16)] = didx_all[pl.ds(off + q * 16, 16)]
        pltpu.sync_copy(hsp_hbm.at[sidx], rows)
        pltpu.sync_copy(asrc_sh.at[sidx], asb)
        pltpu.sync_copy(adst_sh.at[didx], adb)
        for j in range(C // 16):
            av = asb[pl.ds(j * 16, 16)] + adb[pl.ds(j * 16, 16)]
            e = jnp.where(av >= 0.0, av, jnp.float32(0.2) * av)
            ex16 = jnp.exp(e)
            exv_all[pl.ds(off + j * 16, 16)] = ex16
            for k in range(16):
                xs = ex16[k]
                r = j * 16 + k
                for q in range(HHALF // 16):
                    rows[r, pl.ds(q * 16, 16)] = (
                        rows[r, pl.ds(q * 16, 16)] * xs)
        pltpu.sync_copy(rows, u_sh.at[didx], add=True)
        pltpu.sync_copy(exv_all.at[pl.ds(off, C)], s_sh.at[didx], add=True)
        return carry

    lax.fori_loop(0, NCH, chunk, 0)

    @pl.when(c == 0)
    def _():
        pltpu.sync_copy(exv_all, ex_hbm.at[pl.ds(ebase, EPTP)])

    plsc.subcore_barrier()

    @pl.when(tid < 10)
    def _():
        pltpu.sync_copy(u_sh.at[pl.ds(r0, BLK), :], stg)
        pltpu.sync_copy(stg, u_hbm.at[pl.ds(c * N + r0, BLK), :])
        pltpu.sync_copy(s_sh.at[pl.ds(r0, BLK)], stg1)
        pltpu.sync_copy(stg1, s_hbm.at[pl.ds(c * N + r0, BLK)])


def _elu(x):
    return jnp.where(x > 0.0, x, jnp.exp(jnp.minimum(x, 0.0)) - 1.0)


def _tc2_body(u_ref, s_ref, w2_ref, h2_ref, m3sp_ref):
    u = u_ref[...]
    den = s_ref[...] + jnp.float32(1e-16)
    h1 = _elu(jnp.concatenate([u[:N], u[N:]], axis=1) / den)
    h2 = lax.dot_general(h1, w2_ref[...], (((1,), (1,)), ((), ())),
                         preferred_element_type=jnp.float32)
    h2_ref[...] = h2
    m3 = lax.dot_general(h2, w2_ref[...], (((1,), (0,)), ((), ())),
                         preferred_element_type=jnp.float32)
    m3sp_ref[...] = jnp.concatenate([m3[:, :HHALF], m3[:, HHALF:]], axis=0)


def _tc2(u, s, w2):
    return pl.pallas_call(
        _tc2_body,
        out_shape=[
            jax.ShapeDtypeStruct((N, OUT), jnp.float32),
            jax.ShapeDtypeStruct((NC * N, HHALF), jnp.float32),
        ],
    )(u, s, w2)


@functools.partial(
    pl.kernel,
    out_type=[
        jax.ShapeDtypeStruct((NC * N, HHALF), jnp.float32),
    ],
    mesh=_mesh,
    scratch_types=[
        pltpu.VMEM((EPTP,), jnp.int32),
        pltpu.VMEM((EPTP,), jnp.int32),
        pltpu.VMEM((EPTP,), jnp.float32),
        pltpu.VMEM((C,), jnp.int32),
        pltpu.VMEM((C,), jnp.int32),
        pltpu.VMEM((C, HHALF), jnp.float32),
        pltpu.VMEM((BLK, HHALF), jnp.float32),
        pltpu.VMEM_SHARED((NP, HHALF), jnp.float32),
    ],
    compiler_params=_sc_params,
)
def _sc_b(src_hbm, dst_hbm, ex_hbm, m3sp_hbm, z32_hbm,
          v_hbm,
          sidx_all, didx_all, exv_all, sidx, didx, rows, stg, v_sh):
    c = lax.axis_index("c")
    tid = lax.axis_index("s")
    r0 = tid * BLK

    @pl.when(tid < 10)
    def _():
        pltpu.sync_copy(z32_hbm.at[pl.ds(r0, BLK), :], stg)
        pltpu.sync_copy(stg, v_sh.at[pl.ds(r0, BLK), :])

    plsc.subcore_barrier()

    ebase = tid * EPTP
    pltpu.sync_copy(src_hbm.at[pl.ds(ebase, EPTP)], sidx_all)
    pltpu.sync_copy(dst_hbm.at[pl.ds(ebase, EPTP)], didx_all)
    pltpu.sync_copy(ex_hbm.at[pl.ds(ebase, EPTP)], exv_all)
    coff = c * N

    def chunk(i, carry):
        off = i * C
        for q in range(C // 16):
            sidx[pl.ds(q * 16, 16)] = sidx_all[pl.ds(off + q * 16, 16)] + coff
            didx[pl.ds(q * 16, 16)] = didx_all[pl.ds(off + q * 16, 16)]
        pltpu.sync_copy(m3sp_hbm.at[sidx], rows)
        for j in range(C // 16):
            ex16 = exv_all[pl.ds(off + j * 16, 16)]
            for k in range(16):
                xs = ex16[k]
                r = j * 16 + k
                for q in range(HHALF // 16):
                    rows[r, pl.ds(q * 16, 16)] = (
                        rows[r, pl.ds(q * 16, 16)] * xs)
        pltpu.sync_copy(rows, v_sh.at[didx], add=True)
        return carry

    lax.fori_loop(0, NCH, chunk, 0)
    plsc.subcore_barrier()

    @pl.when(tid < 10)
    def _():
        pltpu.sync_copy(v_sh.at[pl.ds(r0, BLK), :], stg)
        pltpu.sync_copy(stg, v_hbm.at[pl.ds(c * N + r0, BLK), :])


def _tc3_body(v_ref, s_ref, w1_ref, h4_ref):
    v = v_ref[...]
    den = s_ref[...] + jnp.float32(1e-16)
    h3 = _elu(jnp.concatenate([v[:N], v[N:]], axis=1) / den)
    h4_ref[...] = lax.dot_general(h3, w1_ref[...], (((1,), (0,)), ((), ())),
                                  preferred_element_type=jnp.float32)


def _tc3(v, s, w1):
    return pl.pallas_call(
        _tc3_body,
        out_shape=jax.ShapeDtypeStruct((N, IN_DIM), jnp.float32),
    )(v, s, w1)


def kernel(g, features, W1, W2, att_src, att_dst):
    src = g[0].astype(jnp.int32)
    dst = g[1].astype(jnp.int32)
    srcp = jnp.concatenate(
        [src.reshape(NS, EPT), jnp.zeros((NS, PAD), jnp.int32)],
        axis=1).reshape(-1)
    dstp = jnp.concatenate(
        [dst.reshape(NS, EPT), jnp.full((NS, PAD), N, jnp.int32)],
        axis=1).reshape(-1)
    z32 = jnp.zeros((N, HHALF), jnp.float32)
    z1 = jnp.zeros((N,), jnp.float32)

    hsp, asrc, adst = _tc1(features, W1, att_src, att_dst)
    ex, u, s2n = _sc_a(srcp, dstp, asrc, adst, hsp, z32, z1)
    s2 = s2n[:N].reshape(N, 1)
    h2, m3sp = _tc2(u, s2, W2)
    (v,) = _sc_b(srcp, dstp, ex, m3sp, z32)
    h4 = _tc3(v, s2, W1)
    return (h2, h4)
